# baseline (speedup 1.0000x reference)
import functools

import jax
import jax.numpy as jnp
from jax import lax
from jax.experimental import pallas as pl
from jax.experimental.pallas import tpu as pltpu
from jax.experimental.pallas import tpu_sc as plsc

_VOCAB = 1000000
_EMB = 64
_HID = 64


_MM_BLK = 4000


def _mm_body(t_ref, w_ref, o_ref):
    o_ref[...] = lax.dot_general(
        t_ref[...], w_ref[...], (((1,), (1,)), ((), ())),
        preferred_element_type=jnp.float32)


def _project(table, W):
    return pl.pallas_call(
        _mm_body,
        grid=(_VOCAB // _MM_BLK,),
        in_specs=[
            pl.BlockSpec((_MM_BLK, _EMB), lambda i: (i, 0)),
            pl.BlockSpec((_HID, _EMB), lambda i: (0, 0)),
        ],
        out_specs=pl.BlockSpec((_MM_BLK, _HID), lambda i: (i, 0)),
        out_shape=jax.ShapeDtypeStruct((_VOCAB, _HID), jnp.float32),
    )(table, W)


_NC, _NS = 2, 16
_NW = _NC * _NS
_TOK = 4096 * 200
_IW = 128
_IROWS = _TOK // _IW
_ROWS_PER_W = _IROWS // _NW
_PER_W = _TOK // _NW
_K = 8
_CHUNK = _K * _IW
_WAVES = _ROWS_PER_W // _K


def _gather_body(proj, pidx, hidx, pout, hout, idx_v, rows_v, sem):
    wid = lax.axis_index("s") * _NC + lax.axis_index("c")
    row0 = wid * _ROWS_PER_W
    out0 = wid * _PER_W
    for idx_hbm, out_hbm in ((pidx, pout), (hidx, hout)):
        def body(w, carry, idx_hbm=idx_hbm, out_hbm=out_hbm):
            pltpu.sync_copy(idx_hbm.at[pl.ds(row0 + w * _K, _K)], idx_v)
            copies = [
                pltpu.async_copy(
                    proj.at[idx_v.at[j]],
                    rows_v.at[pl.ds(j * _IW, _IW)], sem)
                for j in range(_K)
            ]
            for c in copies:
                c.wait()
            pltpu.sync_copy(
                rows_v, out_hbm.at[pl.ds(out0 + w * _CHUNK, _CHUNK)])
            return carry
        lax.fori_loop(0, _WAVES, body, 0)


_gather = functools.partial(
    pl.kernel,
    _gather_body,
    out_type=[jax.ShapeDtypeStruct((_TOK, _HID), jnp.float32)] * 2,
    mesh=plsc.VectorSubcoreMesh(core_axis_name="c", subcore_axis_name="s"),
    scratch_types=[
        pltpu.VMEM((_K, _IW), jnp.int32),
        pltpu.VMEM((_CHUNK, _HID), jnp.float32),
        pltpu.SemaphoreType.DMA,
    ],
    compiler_params=pltpu.CompilerParams(use_tc_tiling_on_sc=False),
)


def kernel(prem, hypo, table, W):
    B, L = prem.shape
    proj = _project(table, W)
    pidx = prem.astype(jnp.int32).reshape(_IROWS, _IW)
    hidx = hypo.astype(jnp.int32).reshape(_IROWS, _IW)
    pout, hout = _gather()(proj, pidx, hidx)
    return pout.reshape(B, L, _HID), hout.reshape(B, L, _HID)


# speedup vs baseline: 1.1946x; 1.1946x over previous
import jax
import jax.numpy as jnp
from jax import lax
from jax.experimental import pallas as pl
from jax.experimental.pallas import tpu as pltpu
from jax.experimental.pallas import tpu_sc as plsc

_VOCAB = 1000000
_EMB = 64
_HID = 64
_PADW = 128
_B = 4096
_L = 200


_MM_BLK = 2000


def _mm_body(t_ref, w_ref, o_ref):
    o_ref[:, :_HID] = lax.dot_general(
        t_ref[...], w_ref[...], (((1,), (1,)), ((), ())),
        preferred_element_type=jnp.float32)
    o_ref[:, _HID:] = jnp.zeros((_MM_BLK, _PADW - _HID), jnp.float32)


def _project(table, W):
    return pl.pallas_call(
        _mm_body,
        grid=(_VOCAB // _MM_BLK,),
        in_specs=[
            pl.BlockSpec((_MM_BLK, _EMB), lambda i: (i, 0)),
            pl.BlockSpec((_HID, _EMB), lambda i: (0, 0)),
        ],
        out_specs=pl.BlockSpec((_MM_BLK, _PADW), lambda i: (i, 0)),
        out_shape=jax.ShapeDtypeStruct((_VOCAB, _PADW), jnp.float32),
    )(table, W)


_NC, _NS = 2, 16
_NW = _NC * _NS
_BATCH_PER_W = _B // _NW
_WB = 2
_WAVES = _BATCH_PER_W // _WB
_SPLITS = ((0, 128), (128, 72))


def _gather_body(proj, pidx, hidx, pout, hout, idx_v, rows_v, sem):
    wid = lax.axis_index("s") * _NC + lax.axis_index("c")
    bbase = wid * _BATCH_PER_W
    for idx_hbm, out_hbm in ((pidx, pout), (hidx, hout)):
        def body(wv, carry, idx_hbm=idx_hbm, out_hbm=out_hbm):
            b0 = bbase + wv * _WB
            pltpu.sync_copy(idx_hbm.at[pl.ds(b0, _WB)], idx_v)
            copies = [
                pltpu.async_copy(
                    proj.at[idx_v.at[i, pl.ds(off, ln)]],
                    rows_v.at[i, pl.ds(off, ln)], sem)
                for i in range(_WB) for off, ln in _SPLITS
            ]
            for c in copies:
                c.wait()
            pltpu.sync_copy(rows_v, out_hbm.at[pl.ds(b0, _WB)])
            return carry
        lax.fori_loop(0, _WAVES, body, 0)


_gather = pl.kernel(
    _gather_body,
    out_type=[jax.ShapeDtypeStruct((_B, _L, _PADW), jnp.float32)] * 2,
    mesh=plsc.VectorSubcoreMesh(core_axis_name="c", subcore_axis_name="s"),
    scratch_types=[
        pltpu.VMEM((_WB, _L), jnp.int32),
        pltpu.VMEM((_WB, _L, _PADW), jnp.float32),
        pltpu.SemaphoreType.DMA,
    ],
    compiler_params=pltpu.CompilerParams(use_tc_tiling_on_sc=False),
)


def kernel(prem, hypo, table, W):
    proj = _project(table, W)
    pout, hout = _gather(proj, prem.astype(jnp.int32), hypo.astype(jnp.int32))
    return pout[:, :, :_HID], hout[:, :, :_HID]
